# Initial kernel scaffold; baseline (speedup 1.0000x reference)
import jax
import jax.numpy as jnp
from jax.experimental import pallas as pl


def kernel(positions, embedding):
    raise NotImplementedError("write your pallas kernel here")


# speedup vs baseline: 5.0388x; 5.0388x over previous
import functools

import jax
import jax.numpy as jnp
from jax import lax
from jax.experimental import pallas as pl
from jax.experimental.pallas import tpu as pltpu
from jax.experimental.pallas import tpu_sc as plsc

MAX_POS = 2048
DEPTH = 4
TABLE_SIZE = (MAX_POS + 1) * DEPTH
TABLE_PAD = 8200

NC = 2
NS = 16
NW = NC * NS

N_IDX = 16384 * 200
PER_W = N_IDX // NW
CHUNK = 3200
NCHUNK = PER_W // CHUNK
GROUPS = CHUNK // 16


def _build():
    mesh = plsc.VectorSubcoreMesh(core_axis_name="c", subcore_axis_name="s")

    @functools.partial(
        pl.kernel,
        mesh=mesh,
        compiler_params=pltpu.CompilerParams(needs_layout_passes=False),
        out_type=jax.ShapeDtypeStruct((N_IDX * DEPTH,), jnp.float32),
        scratch_types=[
            pltpu.VMEM((TABLE_PAD,), jnp.float32),
            pltpu.VMEM((CHUNK,), jnp.int32),
            pltpu.VMEM((CHUNK * DEPTH,), jnp.float32),
        ],
    )
    def k(table_hbm, pos_hbm, out_hbm, table_v, pos_v, out_v):
        wid = lax.axis_index("s") * NC + lax.axis_index("c")
        base = wid * PER_W
        pltpu.sync_copy(table_hbm, table_v)
        iota4 = lax.iota(jnp.int32, 16) * DEPTH

        def chunk_body(c, carry):
            cbase = base + c * CHUNK
            pltpu.sync_copy(pos_hbm.at[pl.ds(cbase, CHUNK)], pos_v)

            def group_body(g, gcarry):
                p = pos_v[pl.ds(g * 16, 16)]
                p = jnp.minimum(jnp.maximum(p, 0), MAX_POS)
                a = p * DEPTH
                ob = g * (16 * DEPTH) + iota4
                for d in range(DEPTH):
                    v = plsc.load_gather(table_v, [a + d])
                    plsc.store_scatter(out_v, [ob + d], v)
                return gcarry

            lax.fori_loop(0, GROUPS, group_body, 0)
            pltpu.sync_copy(out_v, out_hbm.at[pl.ds(cbase * DEPTH, CHUNK * DEPTH)])
            return carry

        lax.fori_loop(0, NCHUNK, chunk_body, 0)

    return k


_sc_lookup = _build()


def kernel(positions, embedding):
    pos_flat = positions.reshape(-1)
    table_flat = jnp.pad(embedding.reshape(-1), (0, TABLE_PAD - TABLE_SIZE))
    out = _sc_lookup(table_flat, pos_flat)
    return out.reshape(positions.shape + (DEPTH,))


# speedup vs baseline: 5.4417x; 1.0800x over previous
import functools

import jax
import jax.numpy as jnp
from jax import lax
from jax.experimental import pallas as pl
from jax.experimental.pallas import tpu as pltpu
from jax.experimental.pallas import tpu_sc as plsc

MAX_POS = 2048
DEPTH = 4
TABLE_SIZE = (MAX_POS + 1) * DEPTH
TABLE_PAD = 8200

NC = 2
NS = 16
NW = NC * NS

N_IDX = 16384 * 200
PER_W = N_IDX // NW
CHUNK = 6400
NCHUNK = PER_W // CHUNK
GROUPS = CHUNK // 16


def _build():
    mesh = plsc.VectorSubcoreMesh(core_axis_name="c", subcore_axis_name="s")

    @functools.partial(
        pl.kernel,
        mesh=mesh,
        compiler_params=pltpu.CompilerParams(needs_layout_passes=False),
        out_type=jax.ShapeDtypeStruct((N_IDX * DEPTH,), jnp.float32),
        scratch_types=[
            pltpu.VMEM((TABLE_PAD,), jnp.float32),
            pltpu.VMEM((CHUNK,), jnp.int32),
            pltpu.VMEM((CHUNK,), jnp.int32),
            pltpu.VMEM((CHUNK * DEPTH,), jnp.float32),
            pltpu.VMEM((CHUNK * DEPTH,), jnp.float32),
            pltpu.SemaphoreType.DMA,
            pltpu.SemaphoreType.DMA,
            pltpu.SemaphoreType.DMA,
            pltpu.SemaphoreType.DMA,
        ],
    )
    def k(table_hbm, pos_hbm, out_hbm, table_v, pos_v0, pos_v1,
          out_v0, out_v1, sin0, sin1, sout0, sout1):
        wid = lax.axis_index("s") * NC + lax.axis_index("c")
        base = wid * PER_W
        pltpu.sync_copy(table_hbm, table_v)
        iota4 = lax.iota(jnp.int32, 16) * DEPTH

        pos_bufs = (pos_v0, pos_v1)
        out_bufs = (out_v0, out_v1)
        sins = (sin0, sin1)
        souts = (sout0, sout1)

        def start_in(c, b):
            pltpu.async_copy(
                pos_hbm.at[pl.ds(base + c * CHUNK, CHUNK)], pos_bufs[b], sins[b])

        def wait_in(b):
            pltpu.make_async_copy(
                pos_hbm.at[pl.ds(base, CHUNK)], pos_bufs[b], sins[b]).wait()

        def start_out(c, b):
            pltpu.async_copy(
                out_bufs[b],
                out_hbm.at[pl.ds((base + c * CHUNK) * DEPTH, CHUNK * DEPTH)],
                souts[b])

        def wait_out(b):
            pltpu.make_async_copy(
                out_bufs[b],
                out_hbm.at[pl.ds(base * DEPTH, CHUNK * DEPTH)],
                souts[b]).wait()

        def compute(b):
            pos_b = pos_bufs[b]
            out_b = out_bufs[b]

            @plsc.parallel_loop(0, GROUPS, unroll=8)
            def _(g):
                p = pos_b[pl.ds(g * 16, 16)]
                p = jnp.minimum(jnp.maximum(p, 0), MAX_POS)
                a = p * DEPTH
                ob = g * (16 * DEPTH) + iota4
                for d in range(DEPTH):
                    v = plsc.load_gather(table_v, [a + d])
                    plsc.store_scatter(out_b, [ob + d], v)

        start_in(0, 0)
        start_in(1, 1)

        def outer(kk, carry):
            for b in range(2):
                c = kk * 2 + b
                wait_in(b)

                @pl.when(kk > 0)
                def _():
                    wait_out(b)

                compute(b)
                start_out(c, b)

                @pl.when(kk < NCHUNK // 2 - 1)
                def _():
                    start_in(c + 2, b)
            return carry

        lax.fori_loop(0, NCHUNK // 2, outer, 0)
        wait_out(0)
        wait_out(1)

    return k


_sc_lookup = _build()


def kernel(positions, embedding):
    pos_flat = positions.reshape(-1)
    table_flat = jnp.pad(embedding.reshape(-1), (0, TABLE_PAD - TABLE_SIZE))
    out = _sc_lookup(table_flat, pos_flat)
    return out.reshape(positions.shape + (DEPTH,))


# speedup vs baseline: 5.4649x; 1.0043x over previous
import functools

import jax
import jax.numpy as jnp
from jax import lax
from jax.experimental import pallas as pl
from jax.experimental.pallas import tpu as pltpu
from jax.experimental.pallas import tpu_sc as plsc

MAX_POS = 2048
DEPTH = 4
TABLE_SIZE = (MAX_POS + 1) * DEPTH
TABLE_PAD = 8200

NC = 2
NS = 16
NW = NC * NS

ROWS = 16384
COLS = 200
ROWS_W = ROWS // NW
CHUNK_ROWS = 32
CHUNK = CHUNK_ROWS * COLS
NCHUNK = ROWS_W // CHUNK_ROWS
GROUPS = CHUNK // 16


def _build():
    mesh = plsc.VectorSubcoreMesh(core_axis_name="c", subcore_axis_name="s")

    @functools.partial(
        pl.kernel,
        mesh=mesh,
        compiler_params=pltpu.CompilerParams(needs_layout_passes=False),
        out_type=jax.ShapeDtypeStruct((ROWS * COLS * DEPTH,), jnp.float32),
        scratch_types=[
            pltpu.VMEM((TABLE_PAD,), jnp.float32),
            pltpu.VMEM((CHUNK_ROWS, COLS), jnp.int32),
            pltpu.VMEM((CHUNK_ROWS, COLS), jnp.int32),
            pltpu.VMEM((CHUNK * DEPTH,), jnp.float32),
            pltpu.VMEM((CHUNK * DEPTH,), jnp.float32),
            pltpu.SemaphoreType.DMA,
            pltpu.SemaphoreType.DMA,
            pltpu.SemaphoreType.DMA,
            pltpu.SemaphoreType.DMA,
        ],
    )
    def k(table_hbm, pos_hbm, out_hbm, table_v, pos_v0, pos_v1,
          out_v0, out_v1, sin0, sin1, sout0, sout1):
        wid = lax.axis_index("s") * NC + lax.axis_index("c")
        row0 = wid * ROWS_W
        pltpu.sync_copy(table_hbm, table_v)
        iota = lax.iota(jnp.int32, 16)
        iota4 = iota * DEPTH
        zero16 = jnp.zeros((16,), jnp.int32)

        pos_bufs = (pos_v0, pos_v1)
        out_bufs = (out_v0, out_v1)
        sins = (sin0, sin1)
        souts = (sout0, sout1)

        def start_in(c, b):
            pltpu.async_copy(
                pos_hbm.at[pl.ds(row0 + c * CHUNK_ROWS, CHUNK_ROWS)],
                pos_bufs[b], sins[b])

        def wait_in(b):
            pltpu.make_async_copy(
                pos_hbm.at[pl.ds(row0, CHUNK_ROWS)], pos_bufs[b], sins[b]).wait()

        def start_out(c, b):
            pltpu.async_copy(
                out_bufs[b],
                out_hbm.at[pl.ds((row0 + c * CHUNK_ROWS) * COLS * DEPTH,
                                 CHUNK * DEPTH)],
                souts[b])

        def wait_out(b):
            pltpu.make_async_copy(
                out_bufs[b],
                out_hbm.at[pl.ds(row0 * COLS * DEPTH, CHUNK * DEPTH)],
                souts[b]).wait()

        def compute(b):
            pos_b = pos_bufs[b]
            out_b = out_bufs[b]

            @plsc.parallel_loop(0, GROUPS, unroll=8)
            def _(g):
                t = g * 16 + iota
                trow = (t * 20972) >> 22
                tcol = t - trow * COLS
                p = plsc.load_gather(pos_b, [trow, tcol])
                p = jnp.minimum(jnp.maximum(p, 0), MAX_POS)
                a = p * DEPTH
                ob = t * DEPTH
                for d in range(DEPTH):
                    v = plsc.load_gather(table_v, [a + d])
                    plsc.store_scatter(out_b, [ob + d], v)

        start_in(0, 0)
        start_in(1, 1)

        def outer(kk, carry):
            for b in range(2):
                c = kk * 2 + b
                wait_in(b)

                @pl.when(kk > 0)
                def _():
                    wait_out(b)

                compute(b)
                start_out(c, b)

                @pl.when(kk < NCHUNK // 2 - 1)
                def _():
                    start_in(c + 2, b)
            return carry

        lax.fori_loop(0, NCHUNK // 2, outer, 0)
        wait_out(0)
        wait_out(1)

    return k


_sc_lookup = _build()


def kernel(positions, embedding):
    table_flat = jnp.pad(embedding.reshape(-1), (0, TABLE_PAD - TABLE_SIZE))
    out = _sc_lookup(table_flat, positions)
    return out.reshape(ROWS, COLS, DEPTH)
